# baseline (speedup 1.0000x reference)
import functools

import jax
import jax.numpy as jnp
from jax import lax
from jax.experimental import pallas as pl
from jax.experimental.pallas import tpu as pltpu
from jax.experimental.pallas import tpu_sc as plsc

N = 10000
E = 320000
D = 128
H = 2
C = 64
NLAT = 64
B = 128

ROW = 144
NH = 10016
NP = 10016
ECHUNK = 64
NTILES = 32
EPT = 10368
CHUNKS = EPT // ECHUNK
EP = EPT * NTILES
NCH = EP // ECHUNK + NTILES
RPT = NP // 16
RB = 1000


def _pre_body(x_ref, w_ref, m_ref, ones_ref, out_ref):
    wm = jnp.dot(w_ref[...], m_ref[...], preferred_element_type=jnp.float32,
        precision=lax.Precision.HIGHEST)
    out_ref[...] = (
        jnp.dot(x_ref[...], wm, preferred_element_type=jnp.float32,
        precision=lax.Precision.HIGHEST)
        + ones_ref[...]
    )


def _pre_call(x, W, M, onesrow):
    return pl.pallas_call(
        _pre_body,
        grid=(N // RB,),
        in_specs=[
            pl.BlockSpec((RB, D), lambda g: (g, 0)),
            pl.BlockSpec((D, D), lambda g: (0, 0)),
            pl.BlockSpec((D, ROW), lambda g: (0, 0)),
            pl.BlockSpec((1, ROW), lambda g: (0, 0)),
        ],
        out_specs=pl.BlockSpec((RB, ROW), lambda g: (g, 0)),
        out_shape=jax.ShapeDtypeStruct((N, ROW), jnp.float32),
    )(x, W, M, onesrow)


def _sc_body(idx_hbm, adst_hbm, haug_hbm, out_hbm,
             acc, adst_l, idx_a, idx_b, rows_a, rows_b, w0_v, w1_v,
             gs_a, gs_b, ss_a, ss_b):
    cid = lax.axis_index("c")
    sid = lax.axis_index("s")
    wid = cid * 16 + sid

    def _zrow(rv):
        def body(i, carry):
            for j in range(ROW // 16):
                rv[i, pl.ds(j * 16, 16)] = jnp.zeros((16,), jnp.float32)
            return carry
        lax.fori_loop(0, ECHUNK, body, 0)

    _zrow(rows_a)
    _zrow(rows_b)
    for k in range(RPT // ECHUNK):
        pltpu.sync_copy(rows_a, acc.at[pl.ds(sid * RPT + k * ECHUNK, ECHUNK)])
    rem = RPT - (RPT // ECHUNK) * ECHUNK
    if rem:
        pltpu.sync_copy(rows_a.at[pl.ds(0, rem)],
                        acc.at[pl.ds(sid * RPT + (RPT // ECHUNK) * ECHUNK, rem)])

    pltpu.sync_copy(adst_hbm, adst_l)

    for g in range(ECHUNK // 16):
        idx_b[1, pl.ds(g * 16, 16)] = jnp.full((16,), N, jnp.int32)

    plsc.subcore_barrier()


    pltpu.sync_copy(idx_hbm.at[wid], idx_a)
    pltpu.async_copy(haug_hbm.at[idx_a.at[0]], rows_a, gs_a)
    pltpu.async_copy(rows_b, acc.at[idx_b.at[1]], ss_b, add=True)

    def _compute(idx_c, rows_c):
        for g in range(ECHUNK // 16):
            e16 = lax.iota(jnp.int32, 16) + g * 16
            c130 = jnp.full((16,), 130, jnp.int32)
            a_s0 = plsc.load_gather(rows_c, [e16, c130])
            a_s1 = plsc.load_gather(rows_c, [e16, c130 + 1])
            d16 = idx_c[1, pl.ds(g * 16, 16)]
            a_d0 = plsc.load_gather(adst_l, [d16 * 2])
            a_d1 = plsc.load_gather(adst_l, [d16 * 2 + 1])
            al0 = a_s0 + a_d0
            al1 = a_s1 + a_d1
            al0 = jnp.where(al0 >= 0.0, al0, al0 * 0.2)
            al1 = jnp.where(al1 >= 0.0, al1, al1 * 0.2)
            w0_v[pl.ds(g * 16, 16)] = jnp.exp(al0)
            w1_v[pl.ds(g * 16, 16)] = jnp.exp(al1)

        sel0 = lax.iota(jnp.int32, 16) == 0

        def _scale16(g, carry):
            w0vec = w0_v[pl.ds(g * 16, 16)]
            w1vec = w1_v[pl.ds(g * 16, 16)]
            for l in range(16):
                e = g * 16 + l
                w0 = w0vec[l]
                w1 = w1vec[l]
                for j in range(4):
                    rows_c[e, pl.ds(j * 16, 16)] = (
                        rows_c[e, pl.ds(j * 16, 16)] * w0)
                for j in range(4, 8):
                    rows_c[e, pl.ds(j * 16, 16)] = (
                        rows_c[e, pl.ds(j * 16, 16)] * w1)
                wv = jnp.where(sel0, w0, w1)
                rows_c[e, pl.ds(128, 16)] = rows_c[e, pl.ds(128, 16)] * wv
            return carry

        lax.fori_loop(0, ECHUNK // 16, _scale16, 0)

    def _step(i, cur, nxt):
        idx_c, rows_c, gs_c, ss_c = cur
        idx_n, rows_n, gs_n, ss_n = nxt
        pltpu.make_async_copy(
            haug_hbm.at[pl.ds(0, ECHUNK)], rows_n, ss_n).wait()
        pltpu.sync_copy(idx_hbm.at[(i + 1) * NTILES + wid], idx_n)
        pltpu.async_copy(haug_hbm.at[idx_n.at[0]], rows_n, gs_n)
        pltpu.make_async_copy(haug_hbm.at[idx_c.at[0]], rows_c, gs_c).wait()
        _compute(idx_c, rows_c)
        pltpu.async_copy(rows_c, acc.at[idx_c.at[1]], ss_c, add=True)

    buf_a = (idx_a, rows_a, gs_a, ss_a)
    buf_b = (idx_b, rows_b, gs_b, ss_b)

    def _pair(k, carry):
        _step(2 * k, buf_a, buf_b)
        _step(2 * k + 1, buf_b, buf_a)
        return carry

    lax.fori_loop(0, CHUNKS // 2, _pair, 0)

    pltpu.make_async_copy(haug_hbm.at[idx_a.at[0]], rows_a, gs_a).wait()
    pltpu.make_async_copy(haug_hbm.at[pl.ds(0, ECHUNK)], rows_b, ss_b).wait()
    plsc.subcore_barrier()

    rbase = sid * RPT
    pltpu.sync_copy(acc.at[pl.ds(rbase, RPT)],
                    out_hbm.at[cid, pl.ds(rbase, RPT)])


def _sc_call(idx2, adst_flat, haug):
    mesh = plsc.VectorSubcoreMesh(core_axis_name="c", subcore_axis_name="s")
    fn = pl.kernel(
        _sc_body,
        out_type=jax.ShapeDtypeStruct((2, NP, ROW), jnp.float32),
        mesh=mesh,
        scratch_types=[
            pltpu.VMEM_SHARED((NP, ROW), jnp.float32),
            pltpu.VMEM((NH * 2,), jnp.float32),
            pltpu.VMEM((2, ECHUNK), jnp.int32),
            pltpu.VMEM((2, ECHUNK), jnp.int32),
            pltpu.VMEM((ECHUNK, ROW), jnp.float32),
            pltpu.VMEM((ECHUNK, ROW), jnp.float32),
            pltpu.VMEM((ECHUNK,), jnp.float32),
            pltpu.VMEM((ECHUNK,), jnp.float32),
            pltpu.SemaphoreType.DMA,
            pltpu.SemaphoreType.DMA,
            pltpu.SemaphoreType.DMA,
            pltpu.SemaphoreType.DMA,
        ],
        compiler_params=pltpu.CompilerParams(
            needs_layout_passes=False, use_tc_tiling_on_sc=False),
    )
    return fn(idx2, adst_flat, haug)


def _post_body(p0_ref, p1_ref, b_ref, bias_ref, fc1w_ref, fc1b_ref,
               out_ref, pooled_acc, cnt_acc):
    g = pl.program_id(0)

    @pl.when(g == 0)
    def _init():
        pooled_acc[...] = jnp.zeros_like(pooled_acc)
        cnt_acc[...] = jnp.zeros_like(cnt_acc)

    blk = p0_ref[0] + p1_ref[0]
    num = blk[:, :D]
    inv0 = 1.0 / (blk[:, 128:129] + 1e-16)
    inv1 = 1.0 / (blk[:, 129:130] + 1e-16)
    inv = jnp.concatenate(
        [jnp.broadcast_to(inv0, (RB, C)), jnp.broadcast_to(inv1, (RB, C))],
        axis=1,
    )
    h2 = jnp.tanh(num * inv + bias_ref[...])

    lanes = lax.broadcasted_iota(jnp.int32, (RB, B), 1).astype(jnp.float32)
    onehot = jnp.where(b_ref[...] == lanes, 1.0, 0.0)
    dn = (((0,), (0,)), ((), ()))
    pooled_acc[...] += lax.dot_general(
        onehot, h2, dn, preferred_element_type=jnp.float32,
        precision=lax.Precision.HIGHEST)
    cnt_acc[...] += lax.dot_general(
        onehot, jnp.ones((RB, B), jnp.float32), dn,
        preferred_element_type=jnp.float32,
        precision=lax.Precision.HIGHEST)

    @pl.when(g == (N // RB) - 1)
    def _fin():
        pooled = pooled_acc[...] / jnp.maximum(cnt_acc[...], 1.0)
        z = jnp.dot(pooled, fc1w_ref[...], preferred_element_type=jnp.float32,
        precision=lax.Precision.HIGHEST)
        out_ref[...] = jnp.tanh(z + fc1b_ref[...])


def _post_call(prt, batch_f, bias_row, fc1_W, fc1b_row):
    return pl.pallas_call(
        _post_body,
        grid=(N // RB,),
        in_specs=[
            pl.BlockSpec((1, RB, ROW), lambda g: (0, g, 0)),
            pl.BlockSpec((1, RB, ROW), lambda g: (1, g, 0)),
            pl.BlockSpec((RB, 1), lambda g: (g, 0)),
            pl.BlockSpec((1, D), lambda g: (0, 0)),
            pl.BlockSpec((D, NLAT), lambda g: (0, 0)),
            pl.BlockSpec((1, NLAT), lambda g: (0, 0)),
        ],
        out_specs=pl.BlockSpec((B, NLAT), lambda g: (0, 0)),
        out_shape=jax.ShapeDtypeStruct((B, NLAT), jnp.float32),
        scratch_shapes=[
            pltpu.VMEM((B, B), jnp.float32),
            pltpu.VMEM((B, B), jnp.float32),
        ],
    )(prt, prt, batch_f, bias_row, fc1_W, fc1b_row)


def kernel(x, edge_index, batch, W, att_src, att_dst, bias, fc1_W, fc1_b):
    M = jnp.zeros((D, ROW), jnp.float32)
    M = M.at[:, :D].set(jnp.eye(D, dtype=jnp.float32))
    M = M.at[0:C, 130].set(att_src[0])
    M = M.at[C:D, 131].set(att_src[1])
    M = M.at[0:C, 132].set(att_dst[0])
    M = M.at[C:D, 133].set(att_dst[1])
    onesrow = jnp.zeros((1, ROW), jnp.float32).at[0, 128:130].set(1.0)

    haug10k = _pre_call(x, W, M, onesrow)
    haug = jnp.pad(haug10k, ((0, NH - N), (0, 0)))
    adst_flat = jnp.pad(
        haug10k[:, 132:134], ((0, NH - N), (0, 0))).reshape(-1)

    sl = jnp.arange(N, dtype=jnp.int32)
    fill = jnp.full((NCH * ECHUNK - E - N,), N, jnp.int32)
    srcpad = jnp.concatenate([edge_index[0], sl, fill]).reshape(NCH, ECHUNK)
    dstpad = jnp.concatenate([edge_index[1], sl, fill]).reshape(NCH, ECHUNK)
    idx2 = jnp.stack([srcpad, dstpad], axis=1)

    prt = _sc_call(idx2, adst_flat, haug)

    batch_f = batch.astype(jnp.float32)[:, None]
    return _post_call(prt, batch_f, bias.reshape(1, D), fc1_W,
                      fc1_b.reshape(1, NLAT))


# speedup vs baseline: 1.1282x; 1.1282x over previous
import functools

import jax
import jax.numpy as jnp
from jax import lax
from jax.experimental import pallas as pl
from jax.experimental.pallas import tpu as pltpu
from jax.experimental.pallas import tpu_sc as plsc

N = 10000
E = 320000
D = 128
H = 2
C = 64
NLAT = 64
B = 128

ROW = 144
NH = 10016
NP = 10016
ECHUNK = 64
NTILES = 32
EPT = 10368
CHUNKS = EPT // ECHUNK
EP = EPT * NTILES
NCH = EP // ECHUNK + NTILES
RPT = NP // 16
RB = 1000


def _pre_body(x_ref, w_ref, m_ref, ones_ref, out_ref):
    wm = jnp.dot(w_ref[...], m_ref[...], preferred_element_type=jnp.float32,
        precision=lax.Precision.HIGHEST)
    out_ref[...] = (
        jnp.dot(x_ref[...], wm, preferred_element_type=jnp.float32,
        precision=lax.Precision.HIGHEST)
        + ones_ref[...]
    )


def _pre_call(x, W, M, onesrow):
    return pl.pallas_call(
        _pre_body,
        grid=(N // RB,),
        in_specs=[
            pl.BlockSpec((RB, D), lambda g: (g, 0)),
            pl.BlockSpec((D, D), lambda g: (0, 0)),
            pl.BlockSpec((D, ROW), lambda g: (0, 0)),
            pl.BlockSpec((1, ROW), lambda g: (0, 0)),
        ],
        out_specs=pl.BlockSpec((RB, ROW), lambda g: (g, 0)),
        out_shape=jax.ShapeDtypeStruct((N, ROW), jnp.float32),
    )(x, W, M, onesrow)


def _sc_body(idx_hbm, adst_hbm, haug_hbm, out_hbm,
             acc, adst_l, idx_a, idx_b, rows_a, rows_b, w0_v, w1_v,
             gs_a, gs_b, ss_a, ss_b):
    cid = lax.axis_index("c")
    sid = lax.axis_index("s")
    wid = cid * 16 + sid

    def _zrow(rv):
        def body(i, carry):
            for j in range(ROW // 16):
                rv[i, pl.ds(j * 16, 16)] = jnp.zeros((16,), jnp.float32)
            return carry
        lax.fori_loop(0, ECHUNK, body, 0)

    _zrow(rows_a)
    _zrow(rows_b)
    for k in range(RPT // ECHUNK):
        pltpu.sync_copy(rows_a, acc.at[pl.ds(sid * RPT + k * ECHUNK, ECHUNK)])
    rem = RPT - (RPT // ECHUNK) * ECHUNK
    if rem:
        pltpu.sync_copy(rows_a.at[pl.ds(0, rem)],
                        acc.at[pl.ds(sid * RPT + (RPT // ECHUNK) * ECHUNK, rem)])

    pltpu.sync_copy(adst_hbm, adst_l)

    for g in range(ECHUNK // 16):
        idx_b[1, pl.ds(g * 16, 16)] = jnp.full((16,), N, jnp.int32)

    plsc.subcore_barrier()

    cbase = wid * CHUNKS

    pltpu.sync_copy(idx_hbm.at[cbase], idx_a)
    pltpu.async_copy(haug_hbm.at[idx_a.at[0]], rows_a, gs_a)
    pltpu.async_copy(rows_b, acc.at[idx_b.at[1]], ss_b, add=True)

    def _compute(idx_c, rows_c):
        for g in range(ECHUNK // 16):
            e16 = lax.iota(jnp.int32, 16) + g * 16
            c130 = jnp.full((16,), 130, jnp.int32)
            a_s0 = plsc.load_gather(rows_c, [e16, c130])
            a_s1 = plsc.load_gather(rows_c, [e16, c130 + 1])
            d16 = idx_c[1, pl.ds(g * 16, 16)]
            a_d0 = plsc.load_gather(adst_l, [d16 * 2])
            a_d1 = plsc.load_gather(adst_l, [d16 * 2 + 1])
            al0 = a_s0 + a_d0
            al1 = a_s1 + a_d1
            al0 = jnp.where(al0 >= 0.0, al0, al0 * 0.2)
            al1 = jnp.where(al1 >= 0.0, al1, al1 * 0.2)
            w0_v[pl.ds(g * 16, 16)] = jnp.exp(al0)
            w1_v[pl.ds(g * 16, 16)] = jnp.exp(al1)

        sel0 = lax.iota(jnp.int32, 16) == 0

        def _scale16(g, carry):
            w0vec = w0_v[pl.ds(g * 16, 16)]
            w1vec = w1_v[pl.ds(g * 16, 16)]
            for l in range(16):
                e = g * 16 + l
                w0 = w0vec[l]
                w1 = w1vec[l]
                for j in range(4):
                    rows_c[e, pl.ds(j * 16, 16)] = (
                        rows_c[e, pl.ds(j * 16, 16)] * w0)
                for j in range(4, 8):
                    rows_c[e, pl.ds(j * 16, 16)] = (
                        rows_c[e, pl.ds(j * 16, 16)] * w1)
                wv = jnp.where(sel0, w0, w1)
                rows_c[e, pl.ds(128, 16)] = rows_c[e, pl.ds(128, 16)] * wv
            return carry

        lax.fori_loop(0, ECHUNK // 16, _scale16, 0)

    def _step(i, cur, nxt):
        idx_c, rows_c, gs_c, ss_c = cur
        idx_n, rows_n, gs_n, ss_n = nxt
        pltpu.make_async_copy(
            haug_hbm.at[pl.ds(0, ECHUNK)], rows_n, ss_n).wait()
        pltpu.sync_copy(idx_hbm.at[cbase + i + 1], idx_n)
        pltpu.async_copy(haug_hbm.at[idx_n.at[0]], rows_n, gs_n)
        pltpu.make_async_copy(haug_hbm.at[idx_c.at[0]], rows_c, gs_c).wait()
        _compute(idx_c, rows_c)
        pltpu.async_copy(rows_c, acc.at[idx_c.at[1]], ss_c, add=True)

    buf_a = (idx_a, rows_a, gs_a, ss_a)
    buf_b = (idx_b, rows_b, gs_b, ss_b)

    def _pair(k, carry):
        _step(2 * k, buf_a, buf_b)
        _step(2 * k + 1, buf_b, buf_a)
        return carry

    lax.fori_loop(0, CHUNKS // 2, _pair, 0)

    pltpu.make_async_copy(haug_hbm.at[idx_a.at[0]], rows_a, gs_a).wait()
    pltpu.make_async_copy(haug_hbm.at[pl.ds(0, ECHUNK)], rows_b, ss_b).wait()
    plsc.subcore_barrier()

    rbase = sid * RPT
    pltpu.sync_copy(acc.at[pl.ds(rbase, RPT)],
                    out_hbm.at[cid, pl.ds(rbase, RPT)])


def _sc_call(idx2, adst_flat, haug):
    mesh = plsc.VectorSubcoreMesh(core_axis_name="c", subcore_axis_name="s")
    fn = pl.kernel(
        _sc_body,
        out_type=jax.ShapeDtypeStruct((2, NP, ROW), jnp.float32),
        mesh=mesh,
        scratch_types=[
            pltpu.VMEM_SHARED((NP, ROW), jnp.float32),
            pltpu.VMEM((NH * 2,), jnp.float32),
            pltpu.VMEM((2, ECHUNK), jnp.int32),
            pltpu.VMEM((2, ECHUNK), jnp.int32),
            pltpu.VMEM((ECHUNK, ROW), jnp.float32),
            pltpu.VMEM((ECHUNK, ROW), jnp.float32),
            pltpu.VMEM((ECHUNK,), jnp.float32),
            pltpu.VMEM((ECHUNK,), jnp.float32),
            pltpu.SemaphoreType.DMA,
            pltpu.SemaphoreType.DMA,
            pltpu.SemaphoreType.DMA,
            pltpu.SemaphoreType.DMA,
        ],
        compiler_params=pltpu.CompilerParams(
            needs_layout_passes=False, use_tc_tiling_on_sc=False),
    )
    return fn(idx2, adst_flat, haug)


def _post_body(p0_ref, p1_ref, b_ref, bias_ref, fc1w_ref, fc1b_ref,
               out_ref, pooled_acc, cnt_acc):
    g = pl.program_id(0)

    @pl.when(g == 0)
    def _init():
        pooled_acc[...] = jnp.zeros_like(pooled_acc)
        cnt_acc[...] = jnp.zeros_like(cnt_acc)

    blk = p0_ref[0] + p1_ref[0]
    num = blk[:, :D]
    inv0 = 1.0 / (blk[:, 128:129] + 1e-16)
    inv1 = 1.0 / (blk[:, 129:130] + 1e-16)
    inv = jnp.concatenate(
        [jnp.broadcast_to(inv0, (RB, C)), jnp.broadcast_to(inv1, (RB, C))],
        axis=1,
    )
    h2 = jnp.tanh(num * inv + bias_ref[...])

    lanes = lax.broadcasted_iota(jnp.int32, (RB, B), 1).astype(jnp.float32)
    onehot = jnp.where(b_ref[...] == lanes, 1.0, 0.0)
    dn = (((0,), (0,)), ((), ()))
    pooled_acc[...] += lax.dot_general(
        onehot, h2, dn, preferred_element_type=jnp.float32,
        precision=lax.Precision.HIGHEST)
    cnt_acc[...] += lax.dot_general(
        onehot, jnp.ones((RB, B), jnp.float32), dn,
        preferred_element_type=jnp.float32,
        precision=lax.Precision.HIGHEST)

    @pl.when(g == (N // RB) - 1)
    def _fin():
        pooled = pooled_acc[...] / jnp.maximum(cnt_acc[...], 1.0)
        z = jnp.dot(pooled, fc1w_ref[...], preferred_element_type=jnp.float32,
        precision=lax.Precision.HIGHEST)
        out_ref[...] = jnp.tanh(z + fc1b_ref[...])


def _post_call(prt, batch_f, bias_row, fc1_W, fc1b_row):
    return pl.pallas_call(
        _post_body,
        grid=(N // RB,),
        in_specs=[
            pl.BlockSpec((1, RB, ROW), lambda g: (0, g, 0)),
            pl.BlockSpec((1, RB, ROW), lambda g: (1, g, 0)),
            pl.BlockSpec((RB, 1), lambda g: (g, 0)),
            pl.BlockSpec((1, D), lambda g: (0, 0)),
            pl.BlockSpec((D, NLAT), lambda g: (0, 0)),
            pl.BlockSpec((1, NLAT), lambda g: (0, 0)),
        ],
        out_specs=pl.BlockSpec((B, NLAT), lambda g: (0, 0)),
        out_shape=jax.ShapeDtypeStruct((B, NLAT), jnp.float32),
        scratch_shapes=[
            pltpu.VMEM((B, B), jnp.float32),
            pltpu.VMEM((B, B), jnp.float32),
        ],
    )(prt, prt, batch_f, bias_row, fc1_W, fc1b_row)


def kernel(x, edge_index, batch, W, att_src, att_dst, bias, fc1_W, fc1_b):
    M = jnp.zeros((D, ROW), jnp.float32)
    M = M.at[:, :D].set(jnp.eye(D, dtype=jnp.float32))
    M = M.at[0:C, 130].set(att_src[0])
    M = M.at[C:D, 131].set(att_src[1])
    M = M.at[0:C, 132].set(att_dst[0])
    M = M.at[C:D, 133].set(att_dst[1])
    onesrow = jnp.zeros((1, ROW), jnp.float32).at[0, 128:130].set(1.0)

    haug10k = _pre_call(x, W, M, onesrow)
    haug = jnp.pad(haug10k, ((0, NH - N), (0, 0)))
    adst_flat = jnp.pad(
        haug10k[:, 132:134], ((0, NH - N), (0, 0))).reshape(-1)

    sl = jnp.arange(N, dtype=jnp.int32)
    fill = jnp.full((NCH * ECHUNK - E - N,), N, jnp.int32)
    srcpad = jnp.concatenate([edge_index[0], sl, fill]).reshape(NCH, ECHUNK)
    dstpad = jnp.concatenate([edge_index[1], sl, fill]).reshape(NCH, ECHUNK)
    idx2 = jnp.stack([srcpad, dstpad], axis=1)

    prt = _sc_call(idx2, adst_flat, haug)

    batch_f = batch.astype(jnp.float32)[:, None]
    return _post_call(prt, batch_f, bias.reshape(1, D), fc1_W,
                      fc1_b.reshape(1, NLAT))


# speedup vs baseline: 1.2196x; 1.0810x over previous
import functools

import jax
import jax.numpy as jnp
from jax import lax
from jax.experimental import pallas as pl
from jax.experimental.pallas import tpu as pltpu
from jax.experimental.pallas import tpu_sc as plsc

N = 10000
E = 320000
D = 128
H = 2
C = 64
NLAT = 64
B = 128

ROW = 144
NH = 10016
NP = 10016
ECHUNK = 64
NTILES = 32
EPT = 10368
CHUNKS = EPT // ECHUNK
EP = EPT * NTILES
NCH = EP // ECHUNK + NTILES
RPT = NP // 16
RB = 1000
SUP = 6
SBS = CHUNKS // SUP


def _pre_body(x_ref, w_ref, m_ref, ones_ref, out_ref):
    wm = jnp.dot(w_ref[...], m_ref[...], preferred_element_type=jnp.float32,
        precision=lax.Precision.HIGHEST)
    out_ref[...] = (
        jnp.dot(x_ref[...], wm, preferred_element_type=jnp.float32,
        precision=lax.Precision.HIGHEST)
        + ones_ref[...]
    )


def _pre_call(x, W, M, onesrow):
    return pl.pallas_call(
        _pre_body,
        grid=(N // RB,),
        in_specs=[
            pl.BlockSpec((RB, D), lambda g: (g, 0)),
            pl.BlockSpec((D, D), lambda g: (0, 0)),
            pl.BlockSpec((D, ROW), lambda g: (0, 0)),
            pl.BlockSpec((1, ROW), lambda g: (0, 0)),
        ],
        out_specs=pl.BlockSpec((RB, ROW), lambda g: (g, 0)),
        out_shape=jax.ShapeDtypeStruct((N, ROW), jnp.float32),
    )(x, W, M, onesrow)


def _sc_body(idx_hbm, adst_hbm, haug_hbm, out_hbm,
             acc, adst_l, isup, rows_a, rows_b, w0_v, w1_v,
             gs_a, gs_b, ss_a, ss_b, isem):
    cid = lax.axis_index("c")
    sid = lax.axis_index("s")
    wid = cid * 16 + sid

    def _zrow(rv):
        def body(i, carry):
            for j in range(ROW // 16):
                rv[i, pl.ds(j * 16, 16)] = jnp.zeros((16,), jnp.float32)
            return carry
        lax.fori_loop(0, ECHUNK, body, 0)

    _zrow(rows_a)
    _zrow(rows_b)
    for k in range(RPT // ECHUNK):
        pltpu.sync_copy(rows_a, acc.at[pl.ds(sid * RPT + k * ECHUNK, ECHUNK)])
    rem = RPT - (RPT // ECHUNK) * ECHUNK
    if rem:
        pltpu.sync_copy(rows_a.at[pl.ds(0, rem)],
                        acc.at[pl.ds(sid * RPT + (RPT // ECHUNK) * ECHUNK, rem)])

    pltpu.sync_copy(adst_hbm, adst_l)

    for g in range(ECHUNK // 16):
        isup[SUP, 1, pl.ds(g * 16, 16)] = jnp.full((16,), N, jnp.int32)

    plsc.subcore_barrier()

    cbase = wid * CHUNKS

    pltpu.sync_copy(idx_hbm.at[pl.ds(cbase, SUP)], isup.at[pl.ds(0, SUP)])
    pltpu.async_copy(haug_hbm.at[isup.at[0, 0]], rows_a, gs_a)
    pltpu.async_copy(rows_b, acc.at[isup.at[SUP, 1]], ss_b, add=True)

    def _compute(ir_c, rows_c):
        for g in range(ECHUNK // 16):
            e16 = lax.iota(jnp.int32, 16) + g * 16
            c130 = jnp.full((16,), 130, jnp.int32)
            a_s0 = plsc.load_gather(rows_c, [e16, c130])
            a_s1 = plsc.load_gather(rows_c, [e16, c130 + 1])
            d16 = isup[ir_c, 1, pl.ds(g * 16, 16)]
            a_d0 = plsc.load_gather(adst_l, [d16 * 2])
            a_d1 = plsc.load_gather(adst_l, [d16 * 2 + 1])
            al0 = a_s0 + a_d0
            al1 = a_s1 + a_d1
            al0 = jnp.where(al0 >= 0.0, al0, al0 * 0.2)
            al1 = jnp.where(al1 >= 0.0, al1, al1 * 0.2)
            w0_v[pl.ds(g * 16, 16)] = jnp.exp(al0)
            w1_v[pl.ds(g * 16, 16)] = jnp.exp(al1)

        sel0 = lax.iota(jnp.int32, 16) == 0

        def _scale16(g, carry):
            w0vec = w0_v[pl.ds(g * 16, 16)]
            w1vec = w1_v[pl.ds(g * 16, 16)]
            for l in range(16):
                e = g * 16 + l
                w0 = w0vec[l]
                w1 = w1vec[l]
                for j in range(4):
                    rows_c[e, pl.ds(j * 16, 16)] = (
                        rows_c[e, pl.ds(j * 16, 16)] * w0)
                for j in range(4, 8):
                    rows_c[e, pl.ds(j * 16, 16)] = (
                        rows_c[e, pl.ds(j * 16, 16)] * w1)
                wv = jnp.where(sel0, w0, w1)
                rows_c[e, pl.ds(128, 16)] = rows_c[e, pl.ds(128, 16)] * wv
            return carry

        lax.fori_loop(0, ECHUNK // 16, _scale16, 0)

    def _step(ir_c, ir_n, cur, nxt):
        rows_c, gs_c, ss_c = cur
        rows_n, gs_n, ss_n = nxt
        pltpu.make_async_copy(
            haug_hbm.at[pl.ds(0, ECHUNK)], rows_n, ss_n).wait()
        pltpu.async_copy(haug_hbm.at[isup.at[ir_n, 0]], rows_n, gs_n)
        pltpu.make_async_copy(haug_hbm.at[isup.at[ir_c, 0]], rows_c,
                              gs_c).wait()
        _compute(ir_c, rows_c)
        pltpu.async_copy(rows_c, acc.at[isup.at[ir_c, 1]], ss_c, add=True)

    buf_a = (rows_a, gs_a, ss_a)
    buf_b = (rows_b, gs_b, ss_b)

    def _sblock(s, carry):
        sb = lax.rem(s, 2) * SUP
        ob = SUP - sb
        for j in range(SUP):
            ir_c = sb + j
            ir_n = sb + j + 1 if j < SUP - 1 else ob
            cur = buf_a if j % 2 == 0 else buf_b
            nxt = buf_b if j % 2 == 0 else buf_a
            _step(ir_c, ir_n, cur, nxt)
            if j == 1:
                pltpu.async_copy(
                    idx_hbm.at[pl.ds(cbase + (s + 1) * SUP, SUP)],
                    isup.at[pl.ds(ob, SUP)], isem)
            if j == 3:
                pltpu.make_async_copy(
                    idx_hbm.at[pl.ds(0, SUP)],
                    isup.at[pl.ds(ob, SUP)], isem).wait()
        return carry

    lax.fori_loop(0, SBS, _sblock, 0)

    pltpu.make_async_copy(haug_hbm.at[isup.at[SUP, 0]], rows_a, gs_a).wait()
    pltpu.make_async_copy(haug_hbm.at[pl.ds(0, ECHUNK)], rows_b, ss_b).wait()
    plsc.subcore_barrier()

    rbase = sid * RPT
    pltpu.sync_copy(acc.at[pl.ds(rbase, RPT)],
                    out_hbm.at[cid, pl.ds(rbase, RPT)])


def _sc_call(idx2, adst_flat, haug):
    mesh = plsc.VectorSubcoreMesh(core_axis_name="c", subcore_axis_name="s")
    fn = pl.kernel(
        _sc_body,
        out_type=jax.ShapeDtypeStruct((2, NP, ROW), jnp.float32),
        mesh=mesh,
        scratch_types=[
            pltpu.VMEM_SHARED((NP, ROW), jnp.float32),
            pltpu.VMEM((NH * 2,), jnp.float32),
            pltpu.VMEM((2 * SUP, 2, ECHUNK), jnp.int32),
            pltpu.VMEM((ECHUNK, ROW), jnp.float32),
            pltpu.VMEM((ECHUNK, ROW), jnp.float32),
            pltpu.VMEM((ECHUNK,), jnp.float32),
            pltpu.VMEM((ECHUNK,), jnp.float32),
            pltpu.SemaphoreType.DMA,
            pltpu.SemaphoreType.DMA,
            pltpu.SemaphoreType.DMA,
            pltpu.SemaphoreType.DMA,
            pltpu.SemaphoreType.DMA,
        ],
        compiler_params=pltpu.CompilerParams(
            needs_layout_passes=False, use_tc_tiling_on_sc=False),
    )
    return fn(idx2, adst_flat, haug)


def _post_body(p0_ref, p1_ref, b_ref, bias_ref, fc1w_ref, fc1b_ref,
               out_ref, pooled_acc, cnt_acc):
    g = pl.program_id(0)

    @pl.when(g == 0)
    def _init():
        pooled_acc[...] = jnp.zeros_like(pooled_acc)
        cnt_acc[...] = jnp.zeros_like(cnt_acc)

    blk = p0_ref[0] + p1_ref[0]
    num = blk[:, :D]
    inv0 = 1.0 / (blk[:, 128:129] + 1e-16)
    inv1 = 1.0 / (blk[:, 129:130] + 1e-16)
    inv = jnp.concatenate(
        [jnp.broadcast_to(inv0, (RB, C)), jnp.broadcast_to(inv1, (RB, C))],
        axis=1,
    )
    h2 = jnp.tanh(num * inv + bias_ref[...])

    lanes = lax.broadcasted_iota(jnp.int32, (RB, B), 1).astype(jnp.float32)
    onehot = jnp.where(b_ref[...] == lanes, 1.0, 0.0)
    dn = (((0,), (0,)), ((), ()))
    pooled_acc[...] += lax.dot_general(
        onehot, h2, dn, preferred_element_type=jnp.float32,
        precision=lax.Precision.HIGHEST)
    cnt_acc[...] += lax.dot_general(
        onehot, jnp.ones((RB, B), jnp.float32), dn,
        preferred_element_type=jnp.float32,
        precision=lax.Precision.HIGHEST)

    @pl.when(g == (N // RB) - 1)
    def _fin():
        pooled = pooled_acc[...] / jnp.maximum(cnt_acc[...], 1.0)
        z = jnp.dot(pooled, fc1w_ref[...], preferred_element_type=jnp.float32,
        precision=lax.Precision.HIGHEST)
        out_ref[...] = jnp.tanh(z + fc1b_ref[...])


def _post_call(prt, batch_f, bias_row, fc1_W, fc1b_row):
    return pl.pallas_call(
        _post_body,
        grid=(N // RB,),
        in_specs=[
            pl.BlockSpec((1, RB, ROW), lambda g: (0, g, 0)),
            pl.BlockSpec((1, RB, ROW), lambda g: (1, g, 0)),
            pl.BlockSpec((RB, 1), lambda g: (g, 0)),
            pl.BlockSpec((1, D), lambda g: (0, 0)),
            pl.BlockSpec((D, NLAT), lambda g: (0, 0)),
            pl.BlockSpec((1, NLAT), lambda g: (0, 0)),
        ],
        out_specs=pl.BlockSpec((B, NLAT), lambda g: (0, 0)),
        out_shape=jax.ShapeDtypeStruct((B, NLAT), jnp.float32),
        scratch_shapes=[
            pltpu.VMEM((B, B), jnp.float32),
            pltpu.VMEM((B, B), jnp.float32),
        ],
    )(prt, prt, batch_f, bias_row, fc1_W, fc1b_row)


def kernel(x, edge_index, batch, W, att_src, att_dst, bias, fc1_W, fc1_b):
    M = jnp.zeros((D, ROW), jnp.float32)
    M = M.at[:, :D].set(jnp.eye(D, dtype=jnp.float32))
    M = M.at[0:C, 130].set(att_src[0])
    M = M.at[C:D, 131].set(att_src[1])
    M = M.at[0:C, 132].set(att_dst[0])
    M = M.at[C:D, 133].set(att_dst[1])
    onesrow = jnp.zeros((1, ROW), jnp.float32).at[0, 128:130].set(1.0)

    haug10k = _pre_call(x, W, M, onesrow)
    haug = jnp.pad(haug10k, ((0, NH - N), (0, 0)))
    adst_flat = jnp.pad(
        haug10k[:, 132:134], ((0, NH - N), (0, 0))).reshape(-1)

    sl = jnp.arange(N, dtype=jnp.int32)
    fill = jnp.full((NCH * ECHUNK - E - N,), N, jnp.int32)
    srcpad = jnp.concatenate([edge_index[0], sl, fill]).reshape(NCH, ECHUNK)
    dstpad = jnp.concatenate([edge_index[1], sl, fill]).reshape(NCH, ECHUNK)
    idx2 = jnp.stack([srcpad, dstpad], axis=1)

    prt = _sc_call(idx2, adst_flat, haug)

    batch_f = batch.astype(jnp.float32)[:, None]
    return _post_call(prt, batch_f, bias.reshape(1, D), fc1_W,
                      fc1_b.reshape(1, NLAT))
